# baseline (speedup 1.0000x reference)
import functools

import jax
import jax.numpy as jnp
from jax import lax
from jax.experimental import pallas as pl
from jax.experimental.pallas import tpu as pltpu
from jax.experimental.pallas import tpu_sc as plsc

VOCAB = 100000
D_MODEL = 128
D_INNER = 256
D_STATE = 16
BATCH = 1024
SEQ = 50


def _sc_gather(tokens_flat, table):
    info = plsc.get_sparse_core_info()
    NC, NS = info.num_cores, info.num_subcores
    NW = NC * NS
    TOT = BATCH * SEQ
    PER_W = TOT // NW
    CH = 80
    NCH = PER_W // CH

    tok3 = tokens_flat.reshape(NW, NCH, CH)
    mesh = plsc.VectorSubcoreMesh(core_axis_name="c", subcore_axis_name="s")

    @functools.partial(
        pl.kernel,
        mesh=mesh,
        out_type=jax.ShapeDtypeStruct((NW, NCH, CH, D_MODEL), jnp.float32),
        scratch_types=[
            pltpu.VMEM((NCH, CH), jnp.int32),
            pltpu.VMEM((CH, D_MODEL), jnp.float32),
            pltpu.VMEM((CH, D_MODEL), jnp.float32),
            pltpu.SemaphoreType.DMA,
            pltpu.SemaphoreType.DMA,
        ],
    )
    def gather_k(tok_hbm, table_hbm, out_hbm, idx_v, buf_a, buf_b, sem_a, sem_b):
        wid = lax.axis_index("s") * NC + lax.axis_index("c")
        pltpu.sync_copy(tok_hbm.at[wid], idx_v)

        def outer(c2, _):
            c0 = c2 * 2
            cp_a = pltpu.async_copy(table_hbm.at[idx_v.at[c0]], buf_a, sem_a)
            cp_b = pltpu.async_copy(table_hbm.at[idx_v.at[c0 + 1]], buf_b, sem_b)
            cp_a.wait()
            pltpu.sync_copy(buf_a, out_hbm.at[wid, c0])
            cp_b.wait()
            pltpu.sync_copy(buf_b, out_hbm.at[wid, c0 + 1])
            return 0

        lax.fori_loop(0, NCH // 2, outer, 0)

    out = gather_k(tok3, table)
    return out.reshape(SEQ, BATCH, D_MODEL)


def _block_last(xT, W_u, b_u, W_dt, b_dt, W_z, b_z, A_log2, W_B_T, W_C, Dv, W_out):
    BC = 256
    G = BATCH // BC

    def body(x_ref, wu_ref, bu_ref, wdt_ref, bdt_ref, wz_ref, bz_ref, alog_ref,
             wbt_ref, wc_ref, d_ref, wout_ref, out_ref, u_scr, dt_scr):
        x3 = x_ref[...]
        x2 = x3.reshape(SEQ * BC, D_MODEL)
        u2 = jnp.dot(x2, wu_ref[...], preferred_element_type=jnp.float32) + bu_ref[...]
        dtr = jnp.dot(x2, wdt_ref[...], preferred_element_type=jnp.float32) + bdt_ref[...]
        dt2 = jnp.maximum(dtr, 0.0) + jnp.log1p(jnp.exp(-jnp.abs(dtr)))

        x_last = x3[SEQ - 1]
        u3 = u2.reshape(SEQ, BC, D_INNER)
        u_last = u3[SEQ - 1]

        MT = jnp.dot(wc_ref[...], wbt_ref[...], preferred_element_type=jnp.float32)
        v = jnp.dot(x_last, MT, preferred_element_type=jnp.float32)
        s3 = jnp.sum(x3 * v[None], axis=-1, keepdims=True)

        u_scr[...] = u3 * s3
        dt_scr[...] = dt2.reshape(SEQ, BC, D_INNER)
        A = -jnp.exp(alog_ref[...])

        def step(k, carry):
            acc, R = carry
            t = SEQ - 1 - k
            dt_t = dt_scr[t]
            su_t = u_scr[t]
            w = dt_t * jnp.exp(A * R)
            return (acc + w * su_t, R + dt_t)

        zeros = jnp.zeros((BC, D_INNER), jnp.float32)
        acc, _ = lax.fori_loop(0, SEQ, step, (zeros, zeros))

        y = acc + d_ref[...] * u_last
        z = jnp.dot(x_last, wz_ref[...], preferred_element_type=jnp.float32) + bz_ref[...]
        e = jnp.exp(-jnp.abs(z))
        sig = jnp.where(z >= 0, 1.0 / (1.0 + e), e / (1.0 + e))
        y = y * (z * sig)
        out_ref[...] = x_last + jnp.dot(y, wout_ref[...],
                                        preferred_element_type=jnp.float32)

    full = lambda shape: pl.BlockSpec(shape, lambda i: (0,) * len(shape))
    return pl.pallas_call(
        body,
        grid=(G,),
        in_specs=[
            pl.BlockSpec((SEQ, BC, D_MODEL), lambda i: (0, i, 0)),
            full((D_MODEL, D_INNER)),
            full((1, D_INNER)),
            full((D_MODEL, D_INNER)),
            full((1, D_INNER)),
            full((D_MODEL, D_INNER)),
            full((1, D_INNER)),
            full((1, D_INNER)),
            full((D_STATE, D_MODEL)),
            full((D_MODEL, D_STATE)),
            full((1, D_INNER)),
            full((D_INNER, D_MODEL)),
        ],
        out_specs=pl.BlockSpec((BC, D_MODEL), lambda i: (i, 0)),
        out_shape=jax.ShapeDtypeStruct((BATCH, D_MODEL), jnp.float32),
        scratch_shapes=[
            pltpu.VMEM((SEQ, BC, D_INNER), jnp.float32),
            pltpu.VMEM((SEQ, BC, D_INNER), jnp.float32),
        ],
    )(xT, W_u, b_u, W_dt, b_dt, W_z, b_z, A_log2, W_B_T, W_C, Dv, W_out)


def _head(xo, W_head):
    VB = 2048
    GV = pl.cdiv(VOCAB, VB)

    def hbody(x_ref, w_ref, o_ref):
        o_ref[...] = jnp.dot(x_ref[...], w_ref[...],
                             preferred_element_type=jnp.float32)

    return pl.pallas_call(
        hbody,
        grid=(GV,),
        in_specs=[
            pl.BlockSpec((BATCH, D_MODEL), lambda j: (0, 0)),
            pl.BlockSpec((D_MODEL, VB), lambda j: (0, j)),
        ],
        out_specs=pl.BlockSpec((BATCH, VB), lambda j: (0, j)),
        out_shape=jax.ShapeDtypeStruct((BATCH, VOCAB), jnp.float32),
    )(xo, W_head)


def kernel(tokens, embed_table, W_in, b_in, W_dt, b_dt, A_log, W_B, W_C, D,
           W_out, W_head):
    tokens_flat = tokens.T.reshape(-1).astype(jnp.int32)
    xT = _sc_gather(tokens_flat, embed_table)

    W_u = W_in[:, :D_INNER]
    W_z = W_in[:, D_INNER:]
    b_u = b_in[:D_INNER].reshape(1, -1)
    b_z = b_in[D_INNER:].reshape(1, -1)
    xo = _block_last(xT, W_u, b_u, W_dt, b_dt.reshape(1, -1), W_z, b_z,
                     A_log.reshape(1, -1), W_B.T, W_C, D.reshape(1, -1), W_out)
    return _head(xo, W_head)


# speedup vs baseline: 1.0024x; 1.0024x over previous
import functools

import jax
import jax.numpy as jnp
from jax import lax
from jax.experimental import pallas as pl
from jax.experimental.pallas import tpu as pltpu
from jax.experimental.pallas import tpu_sc as plsc

VOCAB = 100000
D_MODEL = 128
D_INNER = 256
D_STATE = 16
BATCH = 1024
SEQ = 50


def _sc_gather(tokens_flat, table):
    info = plsc.get_sparse_core_info()
    NC, NS = info.num_cores, info.num_subcores
    NW = NC * NS
    TOT = BATCH * SEQ
    PER_W = TOT // NW
    CH = 80
    NCH = PER_W // CH

    tok3 = tokens_flat.reshape(NW, NCH, CH)
    mesh = plsc.VectorSubcoreMesh(core_axis_name="c", subcore_axis_name="s")

    @functools.partial(
        pl.kernel,
        mesh=mesh,
        out_type=jax.ShapeDtypeStruct((NW, NCH, CH, D_MODEL), jnp.float32),
        scratch_types=[
            pltpu.VMEM((NCH, CH), jnp.int32),
            pltpu.VMEM((CH, D_MODEL), jnp.float32),
            pltpu.VMEM((CH, D_MODEL), jnp.float32),
            pltpu.SemaphoreType.DMA,
            pltpu.SemaphoreType.DMA,
        ],
    )
    def gather_k(tok_hbm, table_hbm, out_hbm, idx_v, buf_a, buf_b, sem_a, sem_b):
        wid = lax.axis_index("s") * NC + lax.axis_index("c")
        pltpu.sync_copy(tok_hbm.at[wid], idx_v)

        def outer(c2, _):
            c0 = c2 * 2
            cp_a = pltpu.async_copy(table_hbm.at[idx_v.at[c0]], buf_a, sem_a)
            cp_b = pltpu.async_copy(table_hbm.at[idx_v.at[c0 + 1]], buf_b, sem_b)
            cp_a.wait()
            pltpu.sync_copy(buf_a, out_hbm.at[wid, c0])
            cp_b.wait()
            pltpu.sync_copy(buf_b, out_hbm.at[wid, c0 + 1])
            return 0

        lax.fori_loop(0, NCH // 2, outer, 0)

    out = gather_k(tok3, table)
    return out.reshape(SEQ, BATCH, D_MODEL)


def _block_last(xT, W_u, b_u, W_dt, b_dt, W_z, b_z, A_log2, W_B_T, W_C, Dv, W_out):
    BC = 256
    G = BATCH // BC

    def body(x_ref, wu_ref, bu_ref, wdt_ref, bdt_ref, wz_ref, bz_ref, alog_ref,
             wbt_ref, wc_ref, d_ref, wout_ref, out_ref, u_scr, dt_scr):
        x3 = x_ref[...]
        x2 = x3.reshape(SEQ * BC, D_MODEL)
        dtr = jnp.dot(x2, wdt_ref[...], preferred_element_type=jnp.float32) + bdt_ref[...]
        dt2 = jnp.maximum(dtr, 0.0) + jnp.log1p(jnp.exp(-jnp.abs(dtr)))
        dt_scr[...] = dt2.reshape(SEQ, BC, D_INNER)

        u2 = jnp.dot(x2, wu_ref[...], preferred_element_type=jnp.float32) + bu_ref[...]
        u3 = u2.reshape(SEQ, BC, D_INNER)
        u_last = u3[SEQ - 1]
        x_last = x3[SEQ - 1]

        MT = jnp.dot(wc_ref[...], wbt_ref[...], preferred_element_type=jnp.float32)
        v = jnp.dot(x_last, MT, preferred_element_type=jnp.float32)
        s3 = jnp.sum(x3 * v[None], axis=-1, keepdims=True)

        A = -jnp.exp(alog_ref[...])
        u_scr[...] = u3 * s3
        u_scr[...] = u_scr[...] * dt_scr[...]
        dt_scr[...] = jnp.exp(A * dt_scr[...])

        def step(k, carry):
            acc, P = carry
            t = SEQ - 2 - k
            P = P * dt_scr[t + 1]
            return (acc + u_scr[t] * P, P)

        ones = jnp.ones((BC, D_INNER), jnp.float32)
        acc, _ = lax.fori_loop(0, SEQ - 1, step, (u_scr[SEQ - 1], ones))

        y = acc + d_ref[...] * u_last
        z = jnp.dot(x_last, wz_ref[...], preferred_element_type=jnp.float32) + bz_ref[...]
        e = jnp.exp(-jnp.abs(z))
        sig = jnp.where(z >= 0, 1.0 / (1.0 + e), e / (1.0 + e))
        y = y * (z * sig)
        out_ref[...] = x_last + jnp.dot(y, wout_ref[...],
                                        preferred_element_type=jnp.float32)

    full = lambda shape: pl.BlockSpec(shape, lambda i: (0,) * len(shape))
    return pl.pallas_call(
        body,
        grid=(G,),
        in_specs=[
            pl.BlockSpec((SEQ, BC, D_MODEL), lambda i: (0, i, 0)),
            full((D_MODEL, D_INNER)),
            full((1, D_INNER)),
            full((D_MODEL, D_INNER)),
            full((1, D_INNER)),
            full((D_MODEL, D_INNER)),
            full((1, D_INNER)),
            full((1, D_INNER)),
            full((D_STATE, D_MODEL)),
            full((D_MODEL, D_STATE)),
            full((1, D_INNER)),
            full((D_INNER, D_MODEL)),
        ],
        out_specs=pl.BlockSpec((BC, D_MODEL), lambda i: (i, 0)),
        out_shape=jax.ShapeDtypeStruct((BATCH, D_MODEL), jnp.float32),
        scratch_shapes=[
            pltpu.VMEM((SEQ, BC, D_INNER), jnp.float32),
            pltpu.VMEM((SEQ, BC, D_INNER), jnp.float32),
        ],
    )(xT, W_u, b_u, W_dt, b_dt, W_z, b_z, A_log2, W_B_T, W_C, Dv, W_out)


def _head(xo, W_head):
    VB = 2048
    GV = pl.cdiv(VOCAB, VB)

    def hbody(x_ref, w_ref, o_ref):
        o_ref[...] = jnp.dot(x_ref[...], w_ref[...],
                             preferred_element_type=jnp.float32)

    return pl.pallas_call(
        hbody,
        grid=(GV,),
        in_specs=[
            pl.BlockSpec((BATCH, D_MODEL), lambda j: (0, 0)),
            pl.BlockSpec((D_MODEL, VB), lambda j: (0, j)),
        ],
        out_specs=pl.BlockSpec((BATCH, VB), lambda j: (0, j)),
        out_shape=jax.ShapeDtypeStruct((BATCH, VOCAB), jnp.float32),
    )(xo, W_head)


def kernel(tokens, embed_table, W_in, b_in, W_dt, b_dt, A_log, W_B, W_C, D,
           W_out, W_head):
    tokens_flat = tokens.T.reshape(-1).astype(jnp.int32)
    xT = _sc_gather(tokens_flat, embed_table)

    W_u = W_in[:, :D_INNER]
    W_z = W_in[:, D_INNER:]
    b_u = b_in[:D_INNER].reshape(1, -1)
    b_z = b_in[D_INNER:].reshape(1, -1)
    xo = _block_last(xT, W_u, b_u, W_dt, b_dt.reshape(1, -1), W_z, b_z,
                     A_log.reshape(1, -1), W_B.T, W_C, D.reshape(1, -1), W_out)
    return _head(xo, W_head)


# speedup vs baseline: 1.0131x; 1.0107x over previous
import functools

import jax
import jax.numpy as jnp
from jax import lax
from jax.experimental import pallas as pl
from jax.experimental.pallas import tpu as pltpu
from jax.experimental.pallas import tpu_sc as plsc

VOCAB = 100000
D_MODEL = 128
D_INNER = 256
D_STATE = 16
BATCH = 1024
SEQ = 50


def _sc_gather(tokens_flat, table):
    info = plsc.get_sparse_core_info()
    NC, NS = info.num_cores, info.num_subcores
    NW = NC * NS
    TOT = tokens_flat.shape[0]
    PER_W = TOT // NW
    CH = 80
    NCH = PER_W // CH

    tok3 = tokens_flat.reshape(NW, NCH, CH)
    mesh = plsc.VectorSubcoreMesh(core_axis_name="c", subcore_axis_name="s")

    @functools.partial(
        pl.kernel,
        mesh=mesh,
        out_type=jax.ShapeDtypeStruct((NW, NCH, CH, D_MODEL), jnp.float32),
        scratch_types=[
            pltpu.VMEM((NCH, CH), jnp.int32),
            pltpu.VMEM((CH, D_MODEL), jnp.float32),
            pltpu.VMEM((CH, D_MODEL), jnp.float32),
            pltpu.SemaphoreType.DMA,
            pltpu.SemaphoreType.DMA,
        ],
    )
    def gather_k(tok_hbm, table_hbm, out_hbm, idx_v, buf_a, buf_b, sem_a, sem_b):
        wid = lax.axis_index("s") * NC + lax.axis_index("c")
        pltpu.sync_copy(tok_hbm.at[wid], idx_v)

        def outer(c2, _):
            c0 = c2 * 2
            cp_a = pltpu.async_copy(table_hbm.at[idx_v.at[c0]], buf_a, sem_a)
            cp_b = pltpu.async_copy(table_hbm.at[idx_v.at[c0 + 1]], buf_b, sem_b)
            cp_a.wait()
            pltpu.sync_copy(buf_a, out_hbm.at[wid, c0])
            cp_b.wait()
            pltpu.sync_copy(buf_b, out_hbm.at[wid, c0 + 1])
            return 0

        lax.fori_loop(0, NCH // 2, outer, 0)

    out = gather_k(tok3, table)
    return out.reshape(SEQ, TOT // SEQ, D_MODEL)


def _block_last(xT, W_u, b_u, W_dt, b_dt, W_z, b_z, A_log2, W_B_T, W_C, Dv, W_out):
    BC = 256
    NB = xT.shape[1]
    G = NB // BC

    def body(x_ref, wu_ref, bu_ref, wdt_ref, bdt_ref, wz_ref, bz_ref, alog_ref,
             wbt_ref, wc_ref, d_ref, wout_ref, out_ref, u_scr, dt_scr):
        x3 = x_ref[...]
        x2 = x3.reshape(SEQ * BC, D_MODEL)
        dtr = jnp.dot(x2, wdt_ref[...], preferred_element_type=jnp.float32) + bdt_ref[...]
        dt2 = jnp.maximum(dtr, 0.0) + jnp.log1p(jnp.exp(-jnp.abs(dtr)))
        dt_scr[...] = dt2.reshape(SEQ, BC, D_INNER)

        u2 = jnp.dot(x2, wu_ref[...], preferred_element_type=jnp.float32) + bu_ref[...]
        u3 = u2.reshape(SEQ, BC, D_INNER)
        u_last = u3[SEQ - 1]
        x_last = x3[SEQ - 1]

        MT = jnp.dot(wc_ref[...], wbt_ref[...], preferred_element_type=jnp.float32)
        v = jnp.dot(x_last, MT, preferred_element_type=jnp.float32)
        s3 = jnp.sum(x3 * v[None], axis=-1, keepdims=True)

        A = -jnp.exp(alog_ref[...])
        u_scr[...] = u3 * s3
        u_scr[...] = u_scr[...] * dt_scr[...]
        dt_scr[...] = jnp.exp(A * dt_scr[...])

        acc = u_scr[SEQ - 1]
        P = dt_scr[SEQ - 1]
        for t in range(SEQ - 2, 0, -1):
            acc = acc + u_scr[t] * P
            P = P * dt_scr[t]
        acc = acc + u_scr[0] * P

        y = acc + d_ref[...] * u_last
        z = jnp.dot(x_last, wz_ref[...], preferred_element_type=jnp.float32) + bz_ref[...]
        e = jnp.exp(-jnp.abs(z))
        sig = jnp.where(z >= 0, 1.0 / (1.0 + e), e / (1.0 + e))
        y = y * (z * sig)
        out_ref[...] = x_last + jnp.dot(y, wout_ref[...],
                                        preferred_element_type=jnp.float32)

    full = lambda shape: pl.BlockSpec(shape, lambda i: (0,) * len(shape))
    return pl.pallas_call(
        body,
        grid=(G,),
        in_specs=[
            pl.BlockSpec((SEQ, BC, D_MODEL), lambda i: (0, i, 0)),
            full((D_MODEL, D_INNER)),
            full((1, D_INNER)),
            full((D_MODEL, D_INNER)),
            full((1, D_INNER)),
            full((D_MODEL, D_INNER)),
            full((1, D_INNER)),
            full((1, D_INNER)),
            full((D_STATE, D_MODEL)),
            full((D_MODEL, D_STATE)),
            full((1, D_INNER)),
            full((D_INNER, D_MODEL)),
        ],
        out_specs=pl.BlockSpec((BC, D_MODEL), lambda i: (i, 0)),
        out_shape=jax.ShapeDtypeStruct((NB, D_MODEL), jnp.float32),
        scratch_shapes=[
            pltpu.VMEM((SEQ, BC, D_INNER), jnp.float32),
            pltpu.VMEM((SEQ, BC, D_INNER), jnp.float32),
        ],
    )(xT, W_u, b_u, W_dt, b_dt, W_z, b_z, A_log2, W_B_T, W_C, Dv, W_out)


def _head(xo, W_head):
    VB = 2048
    GV = pl.cdiv(VOCAB, VB)

    def hbody(x_ref, w_ref, o_ref):
        o_ref[...] = jnp.dot(x_ref[...], w_ref[...],
                             preferred_element_type=jnp.float32)

    return pl.pallas_call(
        hbody,
        grid=(GV,),
        in_specs=[
            pl.BlockSpec((BATCH, D_MODEL), lambda j: (0, 0)),
            pl.BlockSpec((D_MODEL, VB), lambda j: (0, j)),
        ],
        out_specs=pl.BlockSpec((BATCH, VB), lambda j: (0, j)),
        out_shape=jax.ShapeDtypeStruct((BATCH, VOCAB), jnp.float32),
    )(xo, W_head)


def kernel(tokens, embed_table, W_in, b_in, W_dt, b_dt, A_log, W_B, W_C, D,
           W_out, W_head):
    tokens_T = tokens.T.astype(jnp.int32)
    H = BATCH // 2

    W_u = W_in[:, :D_INNER]
    W_z = W_in[:, D_INNER:]
    b_u = b_in[:D_INNER].reshape(1, -1)
    b_z = b_in[D_INNER:].reshape(1, -1)
    wargs = (W_u, b_u, W_dt, b_dt.reshape(1, -1), W_z, b_z,
             A_log.reshape(1, -1), W_B.T, W_C, D.reshape(1, -1), W_out)

    xT0 = _sc_gather(tokens_T[:, :H].reshape(-1), embed_table)
    xT1 = _sc_gather(tokens_T[:, H:].reshape(-1), embed_table)
    xo0 = _block_last(xT0, *wargs)
    xo1 = _block_last(xT1, *wargs)
    xo = jnp.concatenate([xo0, xo1], axis=0)
    return _head(xo, W_head)
